# baseline (speedup 1.0000x reference)
import functools

import jax
import jax.numpy as jnp
from jax import lax
from jax.experimental import pallas as pl
from jax.experimental.pallas import tpu as pltpu
from jax.experimental.pallas import tpu_sc as plsc

_PMT = 16
_D = 1024


def _gather_body(table_hbm, idx_hbm, dep_hbm, out_hbm, idx_s, sem):
    core = lax.axis_index("c")
    nb = out_hbm.shape[0]
    half = nb // 2

    pltpu.sync_copy(idx_hbm, idx_s)
    copies = []
    for j in range(half):
        b = core * half + j
        t = idx_s[b]
        copies.append(
            pltpu.async_copy(table_hbm.at[pl.ds(t, 1)],
                             out_hbm.at[pl.ds(b, 1)], sem))
    for c in copies:
        c.wait()


def _sc_gather(table, idx, dep):
    bs = idx.shape[0]
    mesh = plsc.ScalarSubcoreMesh(axis_name="c", num_cores=2)
    k = functools.partial(
        pl.kernel,
        mesh=mesh,
        out_type=jax.ShapeDtypeStruct((bs, table.shape[1]), jnp.float32),
        scratch_types=[
            pltpu.SMEM((bs,), jnp.int32),
            pltpu.SemaphoreType.DMA,
        ],
    )(_gather_body)
    return k(table, idx, dep)


def _copy_body(src_ref, mask_ref, xo_ref, mo_ref):
    b = pl.program_id(0)
    seq = src_ref.shape[1]
    xo_ref[:, _PMT:, :] = src_ref[:, :seq, :]

    @pl.when(b == 0)
    def _():
        nb = mask_ref.shape[0]
        mo_ref[:, :_PMT] = jnp.zeros((nb, _PMT), mask_ref.dtype)
        mo_ref[:, _PMT:] = mask_ref[...]


def _tc_copy(src_embs, src_mask):
    bs, seq, d = src_embs.shape
    out_seq = seq + _PMT
    return pl.pallas_call(
        _copy_body,
        grid=(bs,),
        in_specs=[
            pl.BlockSpec((1, seq, d), lambda b: (b, 0, 0)),
            pl.BlockSpec((bs, seq), lambda b: (0, 0)),
        ],
        out_specs=[
            pl.BlockSpec((1, out_seq, d), lambda b: (b, 0, 0)),
            pl.BlockSpec((bs, out_seq), lambda b: (0, 0)),
        ],
        out_shape=[
            jax.ShapeDtypeStruct((bs, out_seq, d), src_embs.dtype),
            jax.ShapeDtypeStruct((bs, out_seq), src_mask.dtype),
        ],
    )(src_embs, src_mask)


def _patch_body(p_ref, xin_ref, xo_ref):
    for r in range(_PMT):
        xo_ref[:, r, :] = p_ref[:, pl.ds(r * _D, _D)]


def _tc_patch(p, x):
    bs, out_seq, d = x.shape
    return pl.pallas_call(
        _patch_body,
        grid=(1,),
        in_specs=[
            pl.BlockSpec((bs, _PMT * d), lambda n: (0, 0)),
            pl.BlockSpec(memory_space=pl.ANY),
        ],
        out_specs=pl.BlockSpec((bs, _PMT, d), lambda n: (0, 0, 0)),
        out_shape=jax.ShapeDtypeStruct(x.shape, x.dtype),
        input_output_aliases={1: 0},
    )(p, x)


def kernel(src_embs, src_mask, tissue_idx, prompt_emb):
    x_partial, new_mask = _tc_copy(src_embs, src_mask)
    p = _sc_gather(prompt_emb, tissue_idx, new_mask)
    x = _tc_patch(p, x_partial)
    return (x, new_mask)


# speedup vs baseline: 1.2269x; 1.2269x over previous
import functools

import jax
import jax.numpy as jnp
from jax import lax
from jax.experimental import pallas as pl
from jax.experimental.pallas import tpu as pltpu
from jax.experimental.pallas import tpu_sc as plsc

_PMT = 16
_D = 1024


def _gather_body(table_hbm, idx_hbm, out_hbm, idx_s, sem):
    core = lax.axis_index("c")
    nb = out_hbm.shape[0]
    half = nb // 2

    pltpu.sync_copy(idx_hbm, idx_s)
    copies = []
    for j in range(half):
        b = core * half + j
        t = idx_s[b]
        copies.append(
            pltpu.async_copy(table_hbm.at[pl.ds(t, 1)],
                             out_hbm.at[pl.ds(b, 1)], sem))
    for c in copies:
        c.wait()


def _sc_gather(table, idx):
    bs = idx.shape[0]
    mesh = plsc.ScalarSubcoreMesh(axis_name="c", num_cores=2)
    k = functools.partial(
        pl.kernel,
        mesh=mesh,
        out_type=jax.ShapeDtypeStruct((bs, table.shape[1]), jnp.float32),
        scratch_types=[
            pltpu.SMEM((bs,), jnp.int32),
            pltpu.SemaphoreType.DMA,
        ],
    )(_gather_body)
    return k(table, idx)


def _copy_body(src_ref, mask_ref, xo_ref, mo_ref):
    b = pl.program_id(0)
    seq = src_ref.shape[1]
    xo_ref[:, _PMT:, :] = src_ref[:, :seq, :]

    @pl.when(b == 0)
    def _():
        nb = mask_ref.shape[0]
        mo_ref[:, :_PMT] = jnp.zeros((nb, _PMT), mask_ref.dtype)
        mo_ref[:, _PMT:] = mask_ref[...]


def _tc_copy(src_embs, src_mask):
    bs, seq, d = src_embs.shape
    out_seq = seq + _PMT
    return pl.pallas_call(
        _copy_body,
        grid=(bs,),
        in_specs=[
            pl.BlockSpec((1, seq, d), lambda b: (b, 0, 0)),
            pl.BlockSpec((bs, seq), lambda b: (0, 0)),
        ],
        out_specs=[
            pl.BlockSpec((1, out_seq, d), lambda b: (b, 0, 0)),
            pl.BlockSpec((bs, out_seq), lambda b: (0, 0)),
        ],
        out_shape=[
            jax.ShapeDtypeStruct((bs, out_seq, d), src_embs.dtype),
            jax.ShapeDtypeStruct((bs, out_seq), src_mask.dtype),
        ],
    )(src_embs, src_mask)


def _patch_body(p_ref, xin_ref, xo_ref):
    for r in range(_PMT):
        xo_ref[:, r, :] = p_ref[:, pl.ds(r * _D, _D)]


def _tc_patch(p, x):
    bs, out_seq, d = x.shape
    return pl.pallas_call(
        _patch_body,
        grid=(1,),
        in_specs=[
            pl.BlockSpec((bs, _PMT * d), lambda n: (0, 0)),
            pl.BlockSpec(memory_space=pl.ANY),
        ],
        out_specs=pl.BlockSpec((bs, _PMT, d), lambda n: (0, 0, 0)),
        out_shape=jax.ShapeDtypeStruct(x.shape, x.dtype),
        input_output_aliases={1: 0},
    )(p, x)


def kernel(src_embs, src_mask, tissue_idx, prompt_emb):
    p = _sc_gather(prompt_emb, tissue_idx)
    x_partial, new_mask = _tc_copy(src_embs, src_mask)
    x = _tc_patch(p, x_partial)
    return (x, new_mask)
